# initial kernel scaffold (unmeasured)
import jax
import jax.numpy as jnp
from jax import lax
from jax.experimental import pallas as pl
from jax.experimental.pallas import tpu as pltpu

N_DEV = 4
SQ = 2048
D_MODEL = 1024
HEADS_PER_DEV = 8
DH = 128
QBLK = 256
KWIN = 512
WINDOW = 128
SCALE = 0.08838834764831843
N_QB = SQ // QBLK


def kernel(x, Wq, K_ext, V_ext, Wo):
    my = lax.axis_index("i")
    x_bf = x.reshape(SQ, D_MODEL).astype(jnp.bfloat16)
    wq_bf = Wq.astype(jnp.bfloat16)
    wo_bf = Wo.astype(jnp.bfloat16)
    k_mine = lax.dynamic_index_in_dim(K_ext, my, axis=0, keepdims=False)
    v_mine = lax.dynamic_index_in_dim(V_ext, my, axis=0, keepdims=False)
    k_t = k_mine.astype(jnp.bfloat16).transpose(1, 0, 2)
    v_t = v_mine.astype(jnp.bfloat16).transpose(1, 0, 2)

    def body(x_ref, wq_ref, wo_ref, k_hbm, v_hbm, out_ref,
             wq_buf, wo_buf, k_scr, v_scr, q_scr, ctx_scr,
             wq_send, wq_recv, wo_send, wo_recv, kv_sems):
        me = lax.axis_index("i")
        left = lax.rem(me + N_DEV - 1, N_DEV)
        right = lax.rem(me + 1, N_DEV)

        barrier_sem = pltpu.get_barrier_semaphore()
        for nbr in (left, right):
            pl.semaphore_signal(
                barrier_sem, inc=1,
                device_id=(nbr,), device_id_type=pl.DeviceIdType.MESH,
            )
        pl.semaphore_wait(barrier_sem, 2)

        def attend(cur_wq, cur_wo, first):
            q_scr[:, :] = jnp.dot(
                x_ref[:, :], cur_wq[:, :], preferred_element_type=jnp.float32
            ).astype(jnp.bfloat16)

            def head_body(hh, _):
                def qb_body(qb, _):
                    q = q_scr[pl.ds(qb * QBLK, QBLK), pl.ds(hh * DH, DH)]
                    kstart = jnp.clip(qb * QBLK - WINDOW, 0, SQ - KWIN)
                    kb = k_scr[hh, pl.ds(kstart, KWIN), :]
                    vb = v_scr[hh, pl.ds(kstart, KWIN), :]
                    s = lax.dot_general(
                        q, kb, (((1,), (1,)), ((), ())),
                        preferred_element_type=jnp.float32,
                    ) * SCALE
                    qi = qb * QBLK + lax.broadcasted_iota(jnp.int32, (QBLK, KWIN), 0)
                    ki = kstart + lax.broadcasted_iota(jnp.int32, (QBLK, KWIN), 1)
                    s = jnp.where(jnp.abs(qi - ki) <= WINDOW, s, -1e9)
                    m = jnp.max(s, axis=1, keepdims=True)
                    w = jnp.exp(s - m)
                    w = (w / jnp.sum(w, axis=1, keepdims=True)).astype(jnp.bfloat16)
                    ctx = lax.dot_general(
                        w, vb, (((1,), (0,)), ((), ())),
                        preferred_element_type=jnp.float32,
                    )
                    ctx_scr[pl.ds(qb * QBLK, QBLK), pl.ds(hh * DH, DH)] = (
                        ctx.astype(jnp.bfloat16)
                    )
                    return 0

                lax.fori_loop(0, N_QB, qb_body, 0)
                return 0

            lax.fori_loop(0, HEADS_PER_DEV, head_body, 0)

            partial = jnp.dot(
                ctx_scr[:, :], cur_wo[:, :], preferred_element_type=jnp.float32
            )
            if first:
                out_ref[:, :] = partial
            else:
                out_ref[:, :] = out_ref[:, :] + partial

        for h in range(N_DEV):
            j = lax.rem(me - h + N_DEV, N_DEV)
            if h == 0:
                cur_wq, cur_wo = wq_ref, wo_ref
            else:
                cur_wq, cur_wo = wq_buf.at[h - 1], wo_buf.at[h - 1]

            if h < N_DEV - 1:
                rq = pltpu.make_async_remote_copy(
                    src_ref=cur_wq, dst_ref=wq_buf.at[h],
                    send_sem=wq_send.at[h], recv_sem=wq_recv.at[h],
                    device_id=(right,), device_id_type=pl.DeviceIdType.MESH,
                )
                rw = pltpu.make_async_remote_copy(
                    src_ref=cur_wo, dst_ref=wo_buf.at[h],
                    send_sem=wo_send.at[h], recv_sem=wo_recv.at[h],
                    device_id=(right,), device_id_type=pl.DeviceIdType.MESH,
                )
                rq.start()
                rw.start()

            cp_k = pltpu.make_async_copy(
                k_hbm.at[pl.ds(j * HEADS_PER_DEV, HEADS_PER_DEV)],
                k_scr, kv_sems.at[0],
            )
            cp_v = pltpu.make_async_copy(
                v_hbm.at[pl.ds(j * HEADS_PER_DEV, HEADS_PER_DEV)],
                v_scr, kv_sems.at[1],
            )
            cp_k.start()
            cp_v.start()
            cp_k.wait()
            cp_v.wait()

            attend(cur_wq, cur_wo, first=(h == 0))

            if h < N_DEV - 1:
                rq.wait()
                rw.wait()

    out = pl.pallas_call(
        body,
        out_shape=jax.ShapeDtypeStruct((SQ, D_MODEL), jnp.float32),
        in_specs=[
            pl.BlockSpec(memory_space=pltpu.VMEM),
            pl.BlockSpec(memory_space=pltpu.VMEM),
            pl.BlockSpec(memory_space=pltpu.VMEM),
            pl.BlockSpec(memory_space=pltpu.ANY),
            pl.BlockSpec(memory_space=pltpu.ANY),
        ],
        out_specs=pl.BlockSpec(memory_space=pltpu.VMEM),
        scratch_shapes=[
            pltpu.VMEM((N_DEV - 1, D_MODEL, D_MODEL), jnp.bfloat16),
            pltpu.VMEM((N_DEV - 1, D_MODEL, D_MODEL), jnp.bfloat16),
            pltpu.VMEM((HEADS_PER_DEV, SQ, DH), jnp.bfloat16),
            pltpu.VMEM((HEADS_PER_DEV, SQ, DH), jnp.bfloat16),
            pltpu.VMEM((SQ, D_MODEL), jnp.bfloat16),
            pltpu.VMEM((SQ, D_MODEL), jnp.bfloat16),
            pltpu.SemaphoreType.DMA((N_DEV - 1,)),
            pltpu.SemaphoreType.DMA((N_DEV - 1,)),
            pltpu.SemaphoreType.DMA((N_DEV - 1,)),
            pltpu.SemaphoreType.DMA((N_DEV - 1,)),
            pltpu.SemaphoreType.DMA((2,)),
        ],
        compiler_params=pltpu.CompilerParams(collective_id=0),
    )(x_bf, wq_bf, wo_bf, k_t, v_t)

    return out.reshape(1, SQ, D_MODEL)


# baseline (device time: 302575 ns/iter reference)
import jax
import jax.numpy as jnp
from jax import lax
from jax.experimental import pallas as pl
from jax.experimental.pallas import tpu as pltpu

N_DEV = 4
SQ = 2048
D_MODEL = 1024
HEADS_PER_DEV = 8
DH = 128
QBLK = 256
KWIN = 512
WINDOW = 128
SCALE = 0.08838834764831843
N_QB = SQ // QBLK


def kernel(x, Wq, K_ext, V_ext, Wo):
    my = lax.axis_index("i")
    x_bf = x.reshape(SQ, D_MODEL).astype(jnp.bfloat16)
    wq_bf = Wq.astype(jnp.bfloat16)
    wo_bf = Wo.astype(jnp.bfloat16)
    k_mine = lax.dynamic_index_in_dim(K_ext, my, axis=0, keepdims=False)
    v_mine = lax.dynamic_index_in_dim(V_ext, my, axis=0, keepdims=False)
    k_t = k_mine.astype(jnp.bfloat16).transpose(1, 0, 2)
    v_t = v_mine.astype(jnp.bfloat16).transpose(1, 0, 2)

    def body(x_ref, wq_ref, wo_ref, k_hbm, v_hbm, out_ref,
             wq_buf, wo_buf, k_scr, v_scr, q_scr, ctx_scr,
             wq_send, wq_recv, wo_send, wo_recv, kv_sems):
        me = lax.axis_index("i")
        left = lax.rem(me + N_DEV - 1, N_DEV)
        right = lax.rem(me + 1, N_DEV)

        barrier_sem = pltpu.get_barrier_semaphore()
        for nbr in (left, right):
            pl.semaphore_signal(
                barrier_sem, inc=1,
                device_id=(nbr,), device_id_type=pl.DeviceIdType.MESH,
            )
        pl.semaphore_wait(barrier_sem, 2)

        def attend(cur_wq, cur_wo, first):
            def qproj_body(qb, _):
                rows = pl.ds(qb * QBLK, QBLK)
                q_scr[rows, :] = jnp.dot(
                    x_ref[rows, :], cur_wq[:, :],
                    preferred_element_type=jnp.float32,
                ).astype(jnp.bfloat16)
                return 0

            lax.fori_loop(0, N_QB, qproj_body, 0)

            def head_body(hh, _):
                def qb_body(qb, _):
                    q = q_scr[pl.ds(qb * QBLK, QBLK), pl.ds(hh * DH, DH)]
                    kstart = jnp.clip(qb * QBLK - WINDOW, 0, SQ - KWIN)
                    kstart = pl.multiple_of(kstart, WINDOW)
                    kb = k_scr[hh, pl.ds(kstart, KWIN), :]
                    vb = v_scr[hh, pl.ds(kstart, KWIN), :]
                    s = lax.dot_general(
                        q, kb, (((1,), (1,)), ((), ())),
                        preferred_element_type=jnp.float32,
                    ) * SCALE
                    qi = qb * QBLK + lax.broadcasted_iota(jnp.int32, (QBLK, KWIN), 0)
                    ki = kstart + lax.broadcasted_iota(jnp.int32, (QBLK, KWIN), 1)
                    s = jnp.where(jnp.abs(qi - ki) <= WINDOW, s, -1e9)
                    m = jnp.max(s, axis=1, keepdims=True)
                    w = jnp.exp(s - m)
                    w = (w / jnp.sum(w, axis=1, keepdims=True)).astype(jnp.bfloat16)
                    ctx = lax.dot_general(
                        w, vb, (((1,), (0,)), ((), ())),
                        preferred_element_type=jnp.float32,
                    )
                    ctx_scr[pl.ds(qb * QBLK, QBLK), pl.ds(hh * DH, DH)] = (
                        ctx.astype(jnp.bfloat16)
                    )
                    return 0

                lax.fori_loop(0, N_QB, qb_body, 0)
                return 0

            lax.fori_loop(0, HEADS_PER_DEV, head_body, 0)

            def oproj_body(qb, _):
                rows = pl.ds(qb * QBLK, QBLK)
                partial = jnp.dot(
                    ctx_scr[rows, :], cur_wo[:, :],
                    preferred_element_type=jnp.float32,
                )
                if first:
                    out_ref[rows, :] = partial
                else:
                    out_ref[rows, :] = out_ref[rows, :] + partial
                return 0

            lax.fori_loop(0, N_QB, oproj_body, 0)

        for h in range(N_DEV):
            j = lax.rem(me - h + N_DEV, N_DEV)
            if h == 0:
                cur_wq, cur_wo = wq_ref, wo_ref
            else:
                cur_wq, cur_wo = wq_buf.at[h - 1], wo_buf.at[h - 1]

            if h < N_DEV - 1:
                rq = pltpu.make_async_remote_copy(
                    src_ref=cur_wq, dst_ref=wq_buf.at[h],
                    send_sem=wq_send.at[h], recv_sem=wq_recv.at[h],
                    device_id=(right,), device_id_type=pl.DeviceIdType.MESH,
                )
                rw = pltpu.make_async_remote_copy(
                    src_ref=cur_wo, dst_ref=wo_buf.at[h],
                    send_sem=wo_send.at[h], recv_sem=wo_recv.at[h],
                    device_id=(right,), device_id_type=pl.DeviceIdType.MESH,
                )
                rq.start()
                rw.start()

            cp_k = pltpu.make_async_copy(
                k_hbm.at[pl.ds(j * HEADS_PER_DEV, HEADS_PER_DEV)],
                k_scr, kv_sems.at[0],
            )
            cp_v = pltpu.make_async_copy(
                v_hbm.at[pl.ds(j * HEADS_PER_DEV, HEADS_PER_DEV)],
                v_scr, kv_sems.at[1],
            )
            cp_k.start()
            cp_v.start()
            cp_k.wait()
            cp_v.wait()

            attend(cur_wq, cur_wo, first=(h == 0))

            if h < N_DEV - 1:
                rq.wait()
                rw.wait()

    out = pl.pallas_call(
        body,
        out_shape=jax.ShapeDtypeStruct((SQ, D_MODEL), jnp.float32),
        in_specs=[
            pl.BlockSpec(memory_space=pltpu.VMEM),
            pl.BlockSpec(memory_space=pltpu.VMEM),
            pl.BlockSpec(memory_space=pltpu.VMEM),
            pl.BlockSpec(memory_space=pl.ANY),
            pl.BlockSpec(memory_space=pl.ANY),
        ],
        out_specs=pl.BlockSpec(memory_space=pltpu.VMEM),
        scratch_shapes=[
            pltpu.VMEM((N_DEV - 1, D_MODEL, D_MODEL), jnp.bfloat16),
            pltpu.VMEM((N_DEV - 1, D_MODEL, D_MODEL), jnp.bfloat16),
            pltpu.VMEM((HEADS_PER_DEV, SQ, DH), jnp.bfloat16),
            pltpu.VMEM((HEADS_PER_DEV, SQ, DH), jnp.bfloat16),
            pltpu.VMEM((SQ, D_MODEL), jnp.bfloat16),
            pltpu.VMEM((SQ, D_MODEL), jnp.bfloat16),
            pltpu.SemaphoreType.DMA((N_DEV - 1,)),
            pltpu.SemaphoreType.DMA((N_DEV - 1,)),
            pltpu.SemaphoreType.DMA((N_DEV - 1,)),
            pltpu.SemaphoreType.DMA((N_DEV - 1,)),
            pltpu.SemaphoreType.DMA((2,)),
        ],
        compiler_params=pltpu.CompilerParams(collective_id=0),
    )(x_bf, wq_bf, wo_bf, k_t, v_t)

    return out.reshape(1, SQ, D_MODEL)


# device time: 258925 ns/iter; 1.1686x vs baseline; 1.1686x over previous
import jax
import jax.numpy as jnp
from jax import lax
from jax.experimental import pallas as pl
from jax.experimental.pallas import tpu as pltpu

N_DEV = 4
SQ = 2048
D_MODEL = 1024
HEADS_PER_DEV = 8
DH = 128
QBLK = 256
KWIN = 512
WINDOW = 128
SCALE = 0.08838834764831843
N_QB = SQ // QBLK


def kernel(x, Wq, K_ext, V_ext, Wo):
    my = lax.axis_index("i")
    x_bf = (x.reshape(SQ, D_MODEL) * SCALE).astype(jnp.bfloat16)
    wq_bf = Wq.astype(jnp.bfloat16)
    wo_bf = Wo.astype(jnp.bfloat16)
    k_mine = lax.dynamic_index_in_dim(K_ext, my, axis=0, keepdims=False)
    v_mine = lax.dynamic_index_in_dim(V_ext, my, axis=0, keepdims=False)
    k_t = k_mine.astype(jnp.bfloat16).transpose(1, 0, 2)
    v_t = v_mine.astype(jnp.bfloat16).transpose(1, 0, 2)

    def body(x_ref, wq_ref, wo_ref, k_hbm, v_hbm, out_ref,
             wq_buf, wo_buf, k_scr, v_scr, q_scr, ctx_blk, mask_scr,
             wq_send, wq_recv, wo_send, wo_recv, kv_sems):
        me = lax.axis_index("i")
        left = lax.rem(me + N_DEV - 1, N_DEV)
        right = lax.rem(me + 1, N_DEV)

        barrier_sem = pltpu.get_barrier_semaphore()
        for nbr in (left, right):
            pl.semaphore_signal(
                barrier_sem, inc=1,
                device_id=(nbr,), device_id_type=pl.DeviceIdType.MESH,
            )
        pl.semaphore_wait(barrier_sem, 2)

        def kwin_start(qb):
            ks = jnp.clip(qb * QBLK - WINDOW, 0, SQ - KWIN)
            return pl.multiple_of(ks, WINDOW)

        def start_kv(j):
            cpk = pltpu.make_async_copy(
                k_hbm.at[pl.ds(j * HEADS_PER_DEV, HEADS_PER_DEV)],
                k_scr, kv_sems.at[0],
            )
            cpv = pltpu.make_async_copy(
                v_hbm.at[pl.ds(j * HEADS_PER_DEV, HEADS_PER_DEV)],
                v_scr, kv_sems.at[1],
            )
            cpk.start()
            cpv.start()
            return cpk, cpv

        def mask_body(qb, _):
            ks = kwin_start(qb)
            qi = qb * QBLK + lax.broadcasted_iota(jnp.int32, (QBLK, KWIN), 0)
            ki = ks + lax.broadcasted_iota(jnp.int32, (QBLK, KWIN), 1)
            mask_scr[qb] = jnp.where(
                jnp.abs(qi - ki) <= WINDOW, 0.0, -1e30
            ).astype(jnp.float32)
            return 0

        lax.fori_loop(0, N_QB, mask_body, 0)

        def attend(cur_wq, cur_wo, kv_pending, first):
            def qproj_body(qb, _):
                rows = pl.ds(qb * QBLK, QBLK)
                q_scr[rows, :] = jnp.dot(
                    x_ref[rows, :], cur_wq[:, :],
                    preferred_element_type=jnp.float32,
                ).astype(jnp.bfloat16)
                return 0

            lax.fori_loop(0, N_QB, qproj_body, 0)

            kv_pending[0].wait()
            kv_pending[1].wait()

            def qb_body(qb, _):
                rows = pl.ds(qb * QBLK, QBLK)
                kstart = kwin_start(qb)
                mb = mask_scr[qb]

                def head_body(hh, _):
                    q = q_scr[rows, pl.ds(hh * DH, DH)]
                    kb = k_scr[hh, pl.ds(kstart, KWIN), :]
                    vb = v_scr[hh, pl.ds(kstart, KWIN), :]
                    s = lax.dot_general(
                        q, kb, (((1,), (1,)), ((), ())),
                        preferred_element_type=jnp.float32,
                    ) + mb
                    w = jnp.exp(s)
                    denom = jnp.sum(w, axis=1, keepdims=True)
                    ctx = lax.dot_general(
                        w.astype(jnp.bfloat16), vb, (((1,), (0,)), ((), ())),
                        preferred_element_type=jnp.float32,
                    ) * (1.0 / denom)
                    ctx_blk[:, pl.ds(hh * DH, DH)] = ctx.astype(jnp.bfloat16)
                    return 0

                lax.fori_loop(0, HEADS_PER_DEV, head_body, 0)

                partial = jnp.dot(
                    ctx_blk[:, :], cur_wo[:, :],
                    preferred_element_type=jnp.float32,
                )
                if first:
                    out_ref[rows, :] = partial
                else:
                    out_ref[rows, :] = out_ref[rows, :] + partial
                return 0

            lax.fori_loop(0, N_QB, qb_body, 0)

        for h in range(N_DEV):
            j = lax.rem(me - h + N_DEV, N_DEV)
            if h == 0:
                cur_wq, cur_wo = wq_ref, wo_ref
            else:
                cur_wq, cur_wo = wq_buf.at[h - 1], wo_buf.at[h - 1]

            kv_pending = start_kv(j)

            if h < N_DEV - 1:
                rq = pltpu.make_async_remote_copy(
                    src_ref=cur_wq, dst_ref=wq_buf.at[h],
                    send_sem=wq_send.at[h], recv_sem=wq_recv.at[h],
                    device_id=(right,), device_id_type=pl.DeviceIdType.MESH,
                )
                rw = pltpu.make_async_remote_copy(
                    src_ref=cur_wo, dst_ref=wo_buf.at[h],
                    send_sem=wo_send.at[h], recv_sem=wo_recv.at[h],
                    device_id=(right,), device_id_type=pl.DeviceIdType.MESH,
                )
                rq.start()
                rw.start()

            attend(cur_wq, cur_wo, kv_pending, first=(h == 0))

            if h < N_DEV - 1:
                rq.wait()
                rw.wait()

    out = pl.pallas_call(
        body,
        out_shape=jax.ShapeDtypeStruct((SQ, D_MODEL), jnp.float32),
        in_specs=[
            pl.BlockSpec(memory_space=pltpu.VMEM),
            pl.BlockSpec(memory_space=pltpu.VMEM),
            pl.BlockSpec(memory_space=pltpu.VMEM),
            pl.BlockSpec(memory_space=pl.ANY),
            pl.BlockSpec(memory_space=pl.ANY),
        ],
        out_specs=pl.BlockSpec(memory_space=pltpu.VMEM),
        scratch_shapes=[
            pltpu.VMEM((N_DEV - 1, D_MODEL, D_MODEL), jnp.bfloat16),
            pltpu.VMEM((N_DEV - 1, D_MODEL, D_MODEL), jnp.bfloat16),
            pltpu.VMEM((HEADS_PER_DEV, SQ, DH), jnp.bfloat16),
            pltpu.VMEM((HEADS_PER_DEV, SQ, DH), jnp.bfloat16),
            pltpu.VMEM((SQ, D_MODEL), jnp.bfloat16),
            pltpu.VMEM((QBLK, D_MODEL), jnp.bfloat16),
            pltpu.VMEM((N_QB, QBLK, KWIN), jnp.float32),
            pltpu.SemaphoreType.DMA((N_DEV - 1,)),
            pltpu.SemaphoreType.DMA((N_DEV - 1,)),
            pltpu.SemaphoreType.DMA((N_DEV - 1,)),
            pltpu.SemaphoreType.DMA((N_DEV - 1,)),
            pltpu.SemaphoreType.DMA((2,)),
        ],
        compiler_params=pltpu.CompilerParams(
            collective_id=0,
            vmem_limit_bytes=44 * 1024 * 1024,
        ),
    )(x_bf, wq_bf, wo_bf, k_t, v_t)

    return out.reshape(1, SQ, D_MODEL)


# device time: 208888 ns/iter; 1.4485x vs baseline; 1.2395x over previous
import jax
import jax.numpy as jnp
from jax import lax
from jax.experimental import pallas as pl
from jax.experimental.pallas import tpu as pltpu

N_DEV = 4
SQ = 2048
D_MODEL = 1024
HEADS_PER_DEV = 8
DH = 128
QBLK = 256
KWIN = 512
WINDOW = 128
SCALE = 0.08838834764831843
N_QB = SQ // QBLK


def kernel(x, Wq, K_ext, V_ext, Wo):
    my = lax.axis_index("i")
    x_bf = (x.reshape(SQ, D_MODEL) * SCALE).astype(jnp.bfloat16)
    wq_bf = Wq.astype(jnp.bfloat16)
    wo_bf = Wo.astype(jnp.bfloat16)
    k_mine = lax.dynamic_index_in_dim(K_ext, my, axis=0, keepdims=False)
    v_mine = lax.dynamic_index_in_dim(V_ext, my, axis=0, keepdims=False)
    k_t = k_mine.astype(jnp.bfloat16).transpose(1, 0, 2)
    v_t = v_mine.astype(jnp.bfloat16).transpose(1, 0, 2)

    def body(x_ref, wq_ref, wo_ref, k_hbm, v_hbm, out_ref,
             w_loc, w_fl, w_fr, k_scr, v_scr, q_scr, ctx_blk, mask_scr,
             sr_sems, sl_sem, rl_sems, rr_sem, kv_sems):
        me = lax.axis_index("i")
        left = lax.rem(me + N_DEV - 1, N_DEV)
        right = lax.rem(me + 1, N_DEV)

        w_loc[0, :, :] = wq_ref[:, :]
        w_loc[1, :, :] = wo_ref[:, :]

        barrier_sem = pltpu.get_barrier_semaphore()
        for nbr in (left, right):
            pl.semaphore_signal(
                barrier_sem, inc=1,
                device_id=(nbr,), device_id_type=pl.DeviceIdType.MESH,
            )
        pl.semaphore_wait(barrier_sem, 2)

        def kwin_start(qb):
            ks = jnp.clip(qb * QBLK - WINDOW, 0, SQ - KWIN)
            return pl.multiple_of(ks, WINDOW)

        def start_kv(j):
            cpk = pltpu.make_async_copy(
                k_hbm.at[pl.ds(j * HEADS_PER_DEV, HEADS_PER_DEV)],
                k_scr, kv_sems.at[0],
            )
            cpv = pltpu.make_async_copy(
                v_hbm.at[pl.ds(j * HEADS_PER_DEV, HEADS_PER_DEV)],
                v_scr, kv_sems.at[1],
            )
            cpk.start()
            cpv.start()
            return cpk, cpv

        kv0 = start_kv(me)

        sR = pltpu.make_async_remote_copy(
            src_ref=w_loc, dst_ref=w_fl.at[0],
            send_sem=sr_sems.at[0], recv_sem=rl_sems.at[0],
            device_id=(right,), device_id_type=pl.DeviceIdType.MESH,
        )
        sL = pltpu.make_async_remote_copy(
            src_ref=w_loc, dst_ref=w_fr,
            send_sem=sl_sem.at[0], recv_sem=rr_sem.at[0],
            device_id=(left,), device_id_type=pl.DeviceIdType.MESH,
        )
        sR.start()
        sL.start()

        def mask_body(qb, _):
            ks = kwin_start(qb)
            qi = qb * QBLK + lax.broadcasted_iota(jnp.int32, (QBLK, KWIN), 0)
            ki = ks + lax.broadcasted_iota(jnp.int32, (QBLK, KWIN), 1)
            mask_scr[qb] = jnp.where(
                jnp.abs(qi - ki) <= WINDOW, 0.0, -1e30
            ).astype(jnp.float32)
            return 0

        lax.fori_loop(0, N_QB, mask_body, 0)

        def attend(cur_w, kv_pending, first):
            def qproj_body(qb, _):
                rows = pl.ds(qb * QBLK, QBLK)
                q_scr[rows, :] = jnp.dot(
                    x_ref[rows, :], cur_w[0],
                    preferred_element_type=jnp.float32,
                ).astype(jnp.bfloat16)
                return 0

            lax.fori_loop(0, N_QB, qproj_body, 0)

            kv_pending[0].wait()
            kv_pending[1].wait()

            def qb_body(qb, _):
                rows = pl.ds(qb * QBLK, QBLK)
                kstart = kwin_start(qb)
                mb = mask_scr[qb]

                for hh in range(HEADS_PER_DEV):
                    q = q_scr[rows, pl.ds(hh * DH, DH)]
                    kb = k_scr[hh, pl.ds(kstart, KWIN), :]
                    vb = v_scr[hh, pl.ds(kstart, KWIN), :]
                    s = lax.dot_general(
                        q, kb, (((1,), (1,)), ((), ())),
                        preferred_element_type=jnp.float32,
                    ) + mb
                    w = jnp.exp(s)
                    denom = jnp.sum(w, axis=1, keepdims=True)
                    ctx = lax.dot_general(
                        w.astype(jnp.bfloat16), vb, (((1,), (0,)), ((), ())),
                        preferred_element_type=jnp.float32,
                    ) * (1.0 / denom)
                    ctx_blk[:, pl.ds(hh * DH, DH)] = ctx.astype(jnp.bfloat16)

                partial = jnp.dot(
                    ctx_blk[:, :], cur_w[1],
                    preferred_element_type=jnp.float32,
                )
                if first:
                    out_ref[rows, :] = partial
                else:
                    out_ref[rows, :] = out_ref[rows, :] + partial
                return 0

            lax.fori_loop(0, N_QB, qb_body, 0)

        attend(w_loc, kv0, first=True)

        kv1 = start_kv(lax.rem(me - 1 + N_DEV, N_DEV))
        sR.wait_recv()
        sF = pltpu.make_async_remote_copy(
            src_ref=w_fl.at[0], dst_ref=w_fl.at[1],
            send_sem=sr_sems.at[1], recv_sem=rl_sems.at[1],
            device_id=(right,), device_id_type=pl.DeviceIdType.MESH,
        )
        sF.start()
        attend(w_fl.at[0], kv1, first=False)

        kv2 = start_kv(lax.rem(me + 1, N_DEV))
        sL.wait_recv()
        attend(w_fr, kv2, first=False)

        kv3 = start_kv(lax.rem(me - 2 + N_DEV, N_DEV))
        sF.wait_recv()
        attend(w_fl.at[1], kv3, first=False)

        sR.wait_send()
        sL.wait_send()
        sF.wait_send()

    out = pl.pallas_call(
        body,
        out_shape=jax.ShapeDtypeStruct((SQ, D_MODEL), jnp.float32),
        in_specs=[
            pl.BlockSpec(memory_space=pltpu.VMEM),
            pl.BlockSpec(memory_space=pltpu.VMEM),
            pl.BlockSpec(memory_space=pltpu.VMEM),
            pl.BlockSpec(memory_space=pl.ANY),
            pl.BlockSpec(memory_space=pl.ANY),
        ],
        out_specs=pl.BlockSpec(memory_space=pltpu.VMEM),
        scratch_shapes=[
            pltpu.VMEM((2, D_MODEL, D_MODEL), jnp.bfloat16),
            pltpu.VMEM((2, 2, D_MODEL, D_MODEL), jnp.bfloat16),
            pltpu.VMEM((2, D_MODEL, D_MODEL), jnp.bfloat16),
            pltpu.VMEM((HEADS_PER_DEV, SQ, DH), jnp.bfloat16),
            pltpu.VMEM((HEADS_PER_DEV, SQ, DH), jnp.bfloat16),
            pltpu.VMEM((SQ, D_MODEL), jnp.bfloat16),
            pltpu.VMEM((QBLK, D_MODEL), jnp.bfloat16),
            pltpu.VMEM((N_QB, QBLK, KWIN), jnp.float32),
            pltpu.SemaphoreType.DMA((2,)),
            pltpu.SemaphoreType.DMA((1,)),
            pltpu.SemaphoreType.DMA((2,)),
            pltpu.SemaphoreType.DMA((1,)),
            pltpu.SemaphoreType.DMA((2,)),
        ],
        compiler_params=pltpu.CompilerParams(
            collective_id=0,
            vmem_limit_bytes=44 * 1024 * 1024,
        ),
    )(x_bf, wq_bf, wo_bf, k_t, v_t)

    return out.reshape(1, SQ, D_MODEL)


# device time: 196321 ns/iter; 1.5412x vs baseline; 1.0640x over previous
import jax
import jax.numpy as jnp
from jax import lax
from jax.experimental import pallas as pl
from jax.experimental.pallas import tpu as pltpu

N_DEV = 4
SQ = 2048
D_MODEL = 1024
HEADS_PER_DEV = 8
DH = 128
QBLK = 256
KWIN = 512
WINDOW = 128
SCALE = 0.08838834764831843
N_QB = SQ // QBLK


def kernel(x, Wq, K_ext, V_ext, Wo):
    x_bf = (x.reshape(SQ, D_MODEL) * SCALE).astype(jnp.bfloat16)
    wq_bf = Wq.astype(jnp.bfloat16)
    wo_bf = Wo.astype(jnp.bfloat16)

    def body(x_ref, wq_ref, wo_ref, k_hbm, v_hbm, out_ref,
             w_fl, w_fr, k_scr, v_scr, f_stage,
             q_blk, ctx_blk, mask_scr,
             sr_sems, sl_sems, rl_sems, rr_sems, kv_sems):
        me = lax.axis_index("i")
        left = lax.rem(me + N_DEV - 1, N_DEV)
        right = lax.rem(me + 1, N_DEV)

        barrier_sem = pltpu.get_barrier_semaphore()
        for nbr in (left, right):
            pl.semaphore_signal(
                barrier_sem, inc=1,
                device_id=(nbr,), device_id_type=pl.DeviceIdType.MESH,
            )
        pl.semaphore_wait(barrier_sem, 2)

        def kwin_start(qb):
            ks = jnp.clip(qb * QBLK - WINDOW, 0, SQ - KWIN)
            return pl.multiple_of(ks, WINDOW)

        def start_slice(hbm, j, sem_row):
            cps = []
            for hh in range(HEADS_PER_DEV):
                hd = j * HEADS_PER_DEV + hh
                cp = pltpu.make_async_copy(
                    hbm.at[me, :, hd, :], f_stage.at[hh], kv_sems.at[sem_row, hh],
                )
                cp.start()
                cps.append(cp)
            return cps

        kv_pending = start_slice(k_hbm, me, 0)

        sRq = pltpu.make_async_remote_copy(
            src_ref=wq_ref, dst_ref=w_fl.at[0, 0],
            send_sem=sr_sems.at[0], recv_sem=rl_sems.at[0],
            device_id=(right,), device_id_type=pl.DeviceIdType.MESH,
        )
        sRo = pltpu.make_async_remote_copy(
            src_ref=wo_ref, dst_ref=w_fl.at[0, 1],
            send_sem=sr_sems.at[1], recv_sem=rl_sems.at[1],
            device_id=(right,), device_id_type=pl.DeviceIdType.MESH,
        )
        sLq = pltpu.make_async_remote_copy(
            src_ref=wq_ref, dst_ref=w_fr.at[0],
            send_sem=sl_sems.at[0], recv_sem=rr_sems.at[0],
            device_id=(left,), device_id_type=pl.DeviceIdType.MESH,
        )
        sLo = pltpu.make_async_remote_copy(
            src_ref=wo_ref, dst_ref=w_fr.at[1],
            send_sem=sl_sems.at[1], recv_sem=rr_sems.at[1],
            device_id=(left,), device_id_type=pl.DeviceIdType.MESH,
        )
        sRq.start()
        sRo.start()
        sLq.start()
        sLo.start()

        for idx, qbr in enumerate((0, 1, N_QB - 1)):
            ks = qbr * QBLK - WINDOW
            ks = min(max(ks, 0), SQ - KWIN)
            qi = qbr * QBLK + lax.broadcasted_iota(jnp.int32, (QBLK, KWIN), 0)
            ki = ks + lax.broadcasted_iota(jnp.int32, (QBLK, KWIN), 1)
            mask_scr[idx] = jnp.where(
                jnp.abs(qi - ki) <= WINDOW, 0.0, -1e30
            ).astype(jnp.bfloat16)

        ones_bf = jnp.ones((KWIN, DH), jnp.bfloat16)

        def attend(cur_wq, cur_wo, j, j_next, first):
            for cp in kv_pending_box[0]:
                cp.wait()

            def conv_k(hh, _):
                k_scr[hh, :, :] = f_stage[hh, :, :].astype(jnp.bfloat16)
                return 0

            lax.fori_loop(0, HEADS_PER_DEV, conv_k, 0)
            v_cps = start_slice(v_hbm, j, 1)
            for cp in v_cps:
                cp.wait()

            def conv_v(hh, _):
                v_scr[hh, :, :] = f_stage[hh, :, :].astype(jnp.bfloat16)
                return 0

            lax.fori_loop(0, HEADS_PER_DEV, conv_v, 0)
            if j_next is not None:
                kv_pending_box[0] = start_slice(k_hbm, j_next, 0)

            def qb_body(qb, _):
                rows = pl.ds(qb * QBLK, QBLK)
                kstart = kwin_start(qb)
                midx = jnp.where(qb == 0, 0, jnp.where(qb == N_QB - 1, 2, 1))
                mb = mask_scr[midx].astype(jnp.float32)

                q_blk[:, :] = jnp.dot(
                    x_ref[rows, :], cur_wq[:, :],
                    preferred_element_type=jnp.float32,
                ).astype(jnp.bfloat16)

                def head_pair_body(hp, _):
                    for sub in range(2):
                        hh = hp * 2 + sub
                        q = q_blk[:, pl.ds(hh * DH, DH)]
                        kb = k_scr[hh, pl.ds(kstart, KWIN), :]
                        vb = v_scr[hh, pl.ds(kstart, KWIN), :]
                        s = lax.dot_general(
                            q, kb, (((1,), (1,)), ((), ())),
                            preferred_element_type=jnp.float32,
                        ) + mb
                        wb = jnp.exp(s).astype(jnp.bfloat16)
                        denom = lax.dot_general(
                            wb, ones_bf, (((1,), (0,)), ((), ())),
                            preferred_element_type=jnp.float32,
                        )[:, :1]
                        ctx = lax.dot_general(
                            wb, vb, (((1,), (0,)), ((), ())),
                            preferred_element_type=jnp.float32,
                        ) * (1.0 / denom)
                        ctx_blk[:, pl.ds(hh * DH, DH)] = ctx.astype(jnp.bfloat16)
                    return 0

                lax.fori_loop(0, HEADS_PER_DEV // 2, head_pair_body, 0)

                partial = jnp.dot(
                    ctx_blk[:, :], cur_wo[:, :],
                    preferred_element_type=jnp.float32,
                )
                if first:
                    out_ref[rows, :] = partial
                else:
                    out_ref[rows, :] = out_ref[rows, :] + partial
                return 0

            lax.fori_loop(0, N_QB, qb_body, 0)

        kv_pending_box = [kv_pending]

        attend(wq_ref, wo_ref, me, lax.rem(me - 1 + N_DEV, N_DEV), first=True)

        sRq.wait_recv()
        sRo.wait_recv()
        sF = pltpu.make_async_remote_copy(
            src_ref=w_fl.at[0], dst_ref=w_fl.at[1],
            send_sem=sr_sems.at[2], recv_sem=rl_sems.at[2],
            device_id=(right,), device_id_type=pl.DeviceIdType.MESH,
        )
        sF.start()
        attend(w_fl.at[0, 0], w_fl.at[0, 1],
               lax.rem(me - 1 + N_DEV, N_DEV), lax.rem(me + 1, N_DEV),
               first=False)

        sLq.wait_recv()
        sLo.wait_recv()
        attend(w_fr.at[0], w_fr.at[1],
               lax.rem(me + 1, N_DEV), lax.rem(me - 2 + N_DEV, N_DEV),
               first=False)

        sF.wait_recv()
        attend(w_fl.at[1, 0], w_fl.at[1, 1],
               lax.rem(me - 2 + N_DEV, N_DEV), None, first=False)

        sRq.wait_send()
        sRo.wait_send()
        sLq.wait_send()
        sLo.wait_send()
        sF.wait_send()

    out = pl.pallas_call(
        body,
        out_shape=jax.ShapeDtypeStruct((SQ, D_MODEL), jnp.float32),
        in_specs=[
            pl.BlockSpec(memory_space=pltpu.VMEM),
            pl.BlockSpec(memory_space=pltpu.VMEM),
            pl.BlockSpec(memory_space=pltpu.VMEM),
            pl.BlockSpec(memory_space=pl.ANY),
            pl.BlockSpec(memory_space=pl.ANY),
        ],
        out_specs=pl.BlockSpec(memory_space=pltpu.VMEM),
        scratch_shapes=[
            pltpu.VMEM((2, 2, D_MODEL, D_MODEL), jnp.bfloat16),
            pltpu.VMEM((2, D_MODEL, D_MODEL), jnp.bfloat16),
            pltpu.VMEM((HEADS_PER_DEV, SQ, DH), jnp.bfloat16),
            pltpu.VMEM((HEADS_PER_DEV, SQ, DH), jnp.bfloat16),
            pltpu.VMEM((HEADS_PER_DEV, SQ, DH), jnp.float32),
            pltpu.VMEM((QBLK, D_MODEL), jnp.bfloat16),
            pltpu.VMEM((QBLK, D_MODEL), jnp.bfloat16),
            pltpu.VMEM((3, QBLK, KWIN), jnp.bfloat16),
            pltpu.SemaphoreType.DMA((3,)),
            pltpu.SemaphoreType.DMA((2,)),
            pltpu.SemaphoreType.DMA((3,)),
            pltpu.SemaphoreType.DMA((2,)),
            pltpu.SemaphoreType.DMA((2, HEADS_PER_DEV)),
        ],
        compiler_params=pltpu.CompilerParams(
            collective_id=0,
            vmem_limit_bytes=44 * 1024 * 1024,
        ),
    )(x_bf, wq_bf, wo_bf, K_ext, V_ext)

    return out.reshape(1, SQ, D_MODEL)


# device time: 195543 ns/iter; 1.5474x vs baseline; 1.0040x over previous
import jax
import jax.numpy as jnp
from jax import lax
from jax.experimental import pallas as pl
from jax.experimental.pallas import tpu as pltpu

N_DEV = 4
SQ = 2048
D_MODEL = 1024
HEADS_PER_DEV = 8
DH = 128
QBLK = 256
KWIN = 512
WINDOW = 128
SCALE = 0.08838834764831843
N_QB = SQ // QBLK


def kernel(x, Wq, K_ext, V_ext, Wo):
    x_bf = (x.reshape(SQ, D_MODEL) * SCALE).astype(jnp.bfloat16)
    wq_bf = Wq.astype(jnp.bfloat16)
    wo_bf = Wo.astype(jnp.bfloat16)

    def body(x_ref, wq_ref, wo_ref, k_hbm, v_hbm, out_ref,
             w_fl, w_fr, k_scr, v_scr, f_stage,
             q_blk, ctx_blk, mask_scr,
             sr_sems, sl_sems, rl_sems, rr_sems, kv_sems):
        me = lax.axis_index("i")
        left = lax.rem(me + N_DEV - 1, N_DEV)
        right = lax.rem(me + 1, N_DEV)

        barrier_sem = pltpu.get_barrier_semaphore()
        for nbr in (left, right):
            pl.semaphore_signal(
                barrier_sem, inc=1,
                device_id=(nbr,), device_id_type=pl.DeviceIdType.MESH,
            )
        pl.semaphore_wait(barrier_sem, 2)

        def kwin_start(qb):
            ks = jnp.clip(qb * QBLK - WINDOW, 0, SQ - KWIN)
            return pl.multiple_of(ks, WINDOW)

        def start_slice(hbm, j, sem_row):
            cps = []
            for hh in range(HEADS_PER_DEV):
                hd = j * HEADS_PER_DEV + hh
                cp = pltpu.make_async_copy(
                    hbm.at[me, :, hd, :], f_stage.at[hh], kv_sems.at[sem_row, hh],
                )
                cp.start()
                cps.append(cp)
            return cps

        kv_pending = start_slice(k_hbm, me, 0)

        sRq = pltpu.make_async_remote_copy(
            src_ref=wq_ref, dst_ref=w_fl.at[0, 0],
            send_sem=sr_sems.at[0], recv_sem=rl_sems.at[0],
            device_id=(right,), device_id_type=pl.DeviceIdType.MESH,
        )
        sRo = pltpu.make_async_remote_copy(
            src_ref=wo_ref, dst_ref=w_fl.at[0, 1],
            send_sem=sr_sems.at[1], recv_sem=rl_sems.at[1],
            device_id=(right,), device_id_type=pl.DeviceIdType.MESH,
        )
        sLq = pltpu.make_async_remote_copy(
            src_ref=wq_ref, dst_ref=w_fr.at[0],
            send_sem=sl_sems.at[0], recv_sem=rr_sems.at[0],
            device_id=(left,), device_id_type=pl.DeviceIdType.MESH,
        )
        sLo = pltpu.make_async_remote_copy(
            src_ref=wo_ref, dst_ref=w_fr.at[1],
            send_sem=sl_sems.at[1], recv_sem=rr_sems.at[1],
            device_id=(left,), device_id_type=pl.DeviceIdType.MESH,
        )
        sRq.start()
        sRo.start()
        sLq.start()
        sLo.start()

        for idx, qbr in enumerate((0, 1, N_QB - 1)):
            ks = qbr * QBLK - WINDOW
            ks = min(max(ks, 0), SQ - KWIN)
            qi = qbr * QBLK + lax.broadcasted_iota(jnp.int32, (QBLK, KWIN), 0)
            ki = ks + lax.broadcasted_iota(jnp.int32, (QBLK, KWIN), 1)
            mask_scr[idx] = jnp.where(
                jnp.abs(qi - ki) <= WINDOW, 0.0, -1e30
            ).astype(jnp.bfloat16)

        ones_bf = jnp.ones((KWIN, DH), jnp.bfloat16)

        def attend(cur_wq, cur_wo, j, j_next, first):
            for cp in kv_pending_box[0]:
                cp.wait()

            def conv_k(hh, _):
                k_scr[hh, :, :] = f_stage[hh, :, :].astype(jnp.bfloat16)
                return 0

            lax.fori_loop(0, HEADS_PER_DEV, conv_k, 0)
            v_cps = start_slice(v_hbm, j, 1)
            for cp in v_cps:
                cp.wait()

            def conv_v(hh, _):
                v_scr[hh, :, :] = f_stage[hh, :, :].astype(jnp.bfloat16)
                return 0

            lax.fori_loop(0, HEADS_PER_DEV, conv_v, 0)
            if j_next is not None:
                kv_pending_box[0] = start_slice(k_hbm, j_next, 0)

            def qb_body(qb, _):
                rows = pl.ds(qb * QBLK, QBLK)
                kstart = kwin_start(qb)
                midx = jnp.where(qb == 0, 0, jnp.where(qb == N_QB - 1, 2, 1))
                mb = mask_scr[midx]

                q_blk[:, :] = jnp.dot(
                    x_ref[rows, :], cur_wq[:, :],
                    preferred_element_type=jnp.float32,
                ).astype(jnp.bfloat16)

                def head_pair_body(hp, _):
                    for sub in range(2):
                        hh = hp * 2 + sub
                        q = q_blk[:, pl.ds(hh * DH, DH)]
                        kb = k_scr[hh, pl.ds(kstart, KWIN), :]
                        vb = v_scr[hh, pl.ds(kstart, KWIN), :]
                        s = lax.dot_general(
                            q, kb, (((1,), (1,)), ((), ())),
                            preferred_element_type=jnp.float32,
                        ).astype(jnp.bfloat16) + mb
                        wb = jnp.exp(s)
                        denom = lax.dot_general(
                            wb, ones_bf, (((1,), (0,)), ((), ())),
                            preferred_element_type=jnp.float32,
                        )[:, :1]
                        ctx = lax.dot_general(
                            wb, vb, (((1,), (0,)), ((), ())),
                            preferred_element_type=jnp.float32,
                        ) * (1.0 / denom)
                        ctx_blk[:, pl.ds(hh * DH, DH)] = ctx.astype(jnp.bfloat16)
                    return 0

                lax.fori_loop(0, HEADS_PER_DEV // 2, head_pair_body, 0)

                partial = jnp.dot(
                    ctx_blk[:, :], cur_wo[:, :],
                    preferred_element_type=jnp.float32,
                )
                if first:
                    out_ref[rows, :] = partial
                else:
                    out_ref[rows, :] = out_ref[rows, :] + partial
                return 0

            lax.fori_loop(0, N_QB, qb_body, 0)

        kv_pending_box = [kv_pending]

        attend(wq_ref, wo_ref, me, lax.rem(me - 1 + N_DEV, N_DEV), first=True)

        sRq.wait_recv()
        sRo.wait_recv()
        sF = pltpu.make_async_remote_copy(
            src_ref=w_fl.at[0], dst_ref=w_fl.at[1],
            send_sem=sr_sems.at[2], recv_sem=rl_sems.at[2],
            device_id=(right,), device_id_type=pl.DeviceIdType.MESH,
        )
        sF.start()
        attend(w_fl.at[0, 0], w_fl.at[0, 1],
               lax.rem(me - 1 + N_DEV, N_DEV), lax.rem(me + 1, N_DEV),
               first=False)

        sLq.wait_recv()
        sLo.wait_recv()
        attend(w_fr.at[0], w_fr.at[1],
               lax.rem(me + 1, N_DEV), lax.rem(me - 2 + N_DEV, N_DEV),
               first=False)

        sF.wait_recv()
        attend(w_fl.at[1, 0], w_fl.at[1, 1],
               lax.rem(me - 2 + N_DEV, N_DEV), None, first=False)

        sRq.wait_send()
        sRo.wait_send()
        sLq.wait_send()
        sLo.wait_send()
        sF.wait_send()

    out = pl.pallas_call(
        body,
        out_shape=jax.ShapeDtypeStruct((SQ, D_MODEL), jnp.float32),
        in_specs=[
            pl.BlockSpec(memory_space=pltpu.VMEM),
            pl.BlockSpec(memory_space=pltpu.VMEM),
            pl.BlockSpec(memory_space=pltpu.VMEM),
            pl.BlockSpec(memory_space=pl.ANY),
            pl.BlockSpec(memory_space=pl.ANY),
        ],
        out_specs=pl.BlockSpec(memory_space=pltpu.VMEM),
        scratch_shapes=[
            pltpu.VMEM((2, 2, D_MODEL, D_MODEL), jnp.bfloat16),
            pltpu.VMEM((2, D_MODEL, D_MODEL), jnp.bfloat16),
            pltpu.VMEM((HEADS_PER_DEV, SQ, DH), jnp.bfloat16),
            pltpu.VMEM((HEADS_PER_DEV, SQ, DH), jnp.bfloat16),
            pltpu.VMEM((HEADS_PER_DEV, SQ, DH), jnp.float32),
            pltpu.VMEM((QBLK, D_MODEL), jnp.bfloat16),
            pltpu.VMEM((QBLK, D_MODEL), jnp.bfloat16),
            pltpu.VMEM((3, QBLK, KWIN), jnp.bfloat16),
            pltpu.SemaphoreType.DMA((3,)),
            pltpu.SemaphoreType.DMA((2,)),
            pltpu.SemaphoreType.DMA((3,)),
            pltpu.SemaphoreType.DMA((2,)),
            pltpu.SemaphoreType.DMA((2, HEADS_PER_DEV)),
        ],
        compiler_params=pltpu.CompilerParams(
            collective_id=0,
            vmem_limit_bytes=44 * 1024 * 1024,
        ),
    )(x_bf, wq_bf, wo_bf, K_ext, V_ext)

    return out.reshape(1, SQ, D_MODEL)


# device time: 177327 ns/iter; 1.7063x vs baseline; 1.1027x over previous
import jax
import jax.numpy as jnp
from jax import lax
from jax.experimental import pallas as pl
from jax.experimental.pallas import tpu as pltpu

N_DEV = 4
SQ = 2048
D_MODEL = 1024
HEADS_PER_DEV = 8
DH = 128
QBLK = 256
KWIN = 512
WINDOW = 128
SCALE = 0.08838834764831843
N_QB = SQ // QBLK


def kernel(x, Wq, K_ext, V_ext, Wo):
    x_bf = (x.reshape(SQ, D_MODEL) * SCALE).astype(jnp.bfloat16)
    wq_bf = Wq.astype(jnp.bfloat16)
    wo_bf = Wo.astype(jnp.bfloat16)

    def body(x_ref, wq_ref, wo_ref, k_hbm, v_hbm, out_ref,
             w_fl, w_fr, k_scr, v_scr, f_stage,
             q_scr, ctx_blk, mask_scr,
             sr_sems, sl_sems, rl_sems, rr_sems, kv_sems):
        me = lax.axis_index("i")
        left = lax.rem(me + N_DEV - 1, N_DEV)
        right = lax.rem(me + 1, N_DEV)

        barrier_sem = pltpu.get_barrier_semaphore()
        for nbr in (left, right):
            pl.semaphore_signal(
                barrier_sem, inc=1,
                device_id=(nbr,), device_id_type=pl.DeviceIdType.MESH,
            )
        pl.semaphore_wait(barrier_sem, 2)

        def kwin_start(qb):
            ks = jnp.clip(qb * QBLK - WINDOW, 0, SQ - KWIN)
            return pl.multiple_of(ks, WINDOW)

        def start_slice(hbm, j, sem_row):
            cps = []
            for hh in range(HEADS_PER_DEV):
                hd = j * HEADS_PER_DEV + hh
                cp = pltpu.make_async_copy(
                    hbm.at[me, :, hd, :], f_stage.at[hh], kv_sems.at[sem_row, hh],
                )
                cp.start()
                cps.append(cp)
            return cps

        kv_pending = start_slice(k_hbm, me, 0)

        sRq = pltpu.make_async_remote_copy(
            src_ref=wq_ref, dst_ref=w_fl.at[0, 0],
            send_sem=sr_sems.at[0], recv_sem=rl_sems.at[0],
            device_id=(right,), device_id_type=pl.DeviceIdType.MESH,
        )
        sRo = pltpu.make_async_remote_copy(
            src_ref=wo_ref, dst_ref=w_fl.at[0, 1],
            send_sem=sr_sems.at[1], recv_sem=rl_sems.at[1],
            device_id=(right,), device_id_type=pl.DeviceIdType.MESH,
        )
        sLq = pltpu.make_async_remote_copy(
            src_ref=wq_ref, dst_ref=w_fr.at[0],
            send_sem=sl_sems.at[0], recv_sem=rr_sems.at[0],
            device_id=(left,), device_id_type=pl.DeviceIdType.MESH,
        )
        sLo = pltpu.make_async_remote_copy(
            src_ref=wo_ref, dst_ref=w_fr.at[1],
            send_sem=sl_sems.at[1], recv_sem=rr_sems.at[1],
            device_id=(left,), device_id_type=pl.DeviceIdType.MESH,
        )
        sRq.start()
        sRo.start()
        sLq.start()
        sLo.start()

        for idx, qbr in enumerate((0, 1, N_QB - 1)):
            ks = qbr * QBLK - WINDOW
            ks = min(max(ks, 0), SQ - KWIN)
            qi = qbr * QBLK + lax.broadcasted_iota(jnp.int32, (QBLK, KWIN), 0)
            ki = ks + lax.broadcasted_iota(jnp.int32, (QBLK, KWIN), 1)
            mask_scr[idx] = jnp.where(
                jnp.abs(qi - ki) <= WINDOW, 0.0, -1e30
            ).astype(jnp.bfloat16)

        ones_bf = jnp.ones((KWIN, DH), jnp.bfloat16)

        def attend(cur_wq, cur_wo, j, j_next, first):
            for cp in kv_pending_box[0]:
                cp.wait()

            def conv_k(hh, _):
                k_scr[hh, :, :] = f_stage[hh, :, :].astype(jnp.bfloat16)
                return 0

            lax.fori_loop(0, HEADS_PER_DEV, conv_k, 0)
            v_cps = start_slice(v_hbm, j, 1)

            def qproj_body(qb, _):
                rows = pl.ds(qb * QBLK, QBLK)
                q_scr[rows, :] = jnp.dot(
                    x_ref[rows, :], cur_wq[:, :],
                    preferred_element_type=jnp.float32,
                ).astype(jnp.bfloat16)
                return 0

            lax.fori_loop(0, N_QB, qproj_body, 0)

            for cp in v_cps:
                cp.wait()

            def conv_v(hh, _):
                v_scr[hh, :, :] = f_stage[hh, :, :].astype(jnp.bfloat16)
                return 0

            lax.fori_loop(0, HEADS_PER_DEV, conv_v, 0)
            if j_next is not None:
                kv_pending_box[0] = start_slice(k_hbm, j_next, 0)

            def qb_body(qb, _):
                rows = pl.ds(qb * QBLK, QBLK)
                kstart = kwin_start(qb)
                midx = jnp.where(qb == 0, 0, jnp.where(qb == N_QB - 1, 2, 1))
                mb = mask_scr[midx]

                def head_quad_body(hp, _):
                    for sub in range(4):
                        hh = hp * 4 + sub
                        q = q_scr[rows, pl.ds(hh * DH, DH)]
                        kb = k_scr[hh, pl.ds(kstart, KWIN), :]
                        vb = v_scr[hh, pl.ds(kstart, KWIN), :]
                        s = lax.dot_general(
                            q, kb, (((1,), (1,)), ((), ())),
                            preferred_element_type=jnp.float32,
                        ).astype(jnp.bfloat16) + mb
                        wb = jnp.exp(s)
                        denom = lax.dot_general(
                            wb, ones_bf, (((1,), (0,)), ((), ())),
                            preferred_element_type=jnp.float32,
                        )[:, :1]
                        ctx = lax.dot_general(
                            wb, vb, (((1,), (0,)), ((), ())),
                            preferred_element_type=jnp.float32,
                        ) * (1.0 / denom)
                        ctx_blk[:, pl.ds(hh * DH, DH)] = ctx.astype(jnp.bfloat16)
                    return 0

                lax.fori_loop(0, HEADS_PER_DEV // 4, head_quad_body, 0)

                partial = jnp.dot(
                    ctx_blk[:, :], cur_wo[:, :],
                    preferred_element_type=jnp.float32,
                )
                if first:
                    out_ref[rows, :] = partial
                else:
                    out_ref[rows, :] = out_ref[rows, :] + partial
                return 0

            lax.fori_loop(0, N_QB, qb_body, 0)

        kv_pending_box = [kv_pending]

        attend(wq_ref, wo_ref, me, lax.rem(me - 1 + N_DEV, N_DEV), first=True)

        sRq.wait_recv()
        sRo.wait_recv()
        sF = pltpu.make_async_remote_copy(
            src_ref=w_fl.at[0], dst_ref=w_fl.at[1],
            send_sem=sr_sems.at[2], recv_sem=rl_sems.at[2],
            device_id=(right,), device_id_type=pl.DeviceIdType.MESH,
        )
        sF.start()
        attend(w_fl.at[0, 0], w_fl.at[0, 1],
               lax.rem(me - 1 + N_DEV, N_DEV), lax.rem(me + 1, N_DEV),
               first=False)

        sLq.wait_recv()
        sLo.wait_recv()
        attend(w_fr.at[0], w_fr.at[1],
               lax.rem(me + 1, N_DEV), lax.rem(me - 2 + N_DEV, N_DEV),
               first=False)

        sF.wait_recv()
        attend(w_fl.at[1, 0], w_fl.at[1, 1],
               lax.rem(me - 2 + N_DEV, N_DEV), None, first=False)

        sRq.wait_send()
        sRo.wait_send()
        sLq.wait_send()
        sLo.wait_send()
        sF.wait_send()

    out = pl.pallas_call(
        body,
        out_shape=jax.ShapeDtypeStruct((SQ, D_MODEL), jnp.float32),
        in_specs=[
            pl.BlockSpec(memory_space=pltpu.VMEM),
            pl.BlockSpec(memory_space=pltpu.VMEM),
            pl.BlockSpec(memory_space=pltpu.VMEM),
            pl.BlockSpec(memory_space=pl.ANY),
            pl.BlockSpec(memory_space=pl.ANY),
        ],
        out_specs=pl.BlockSpec(memory_space=pltpu.VMEM),
        scratch_shapes=[
            pltpu.VMEM((2, 2, D_MODEL, D_MODEL), jnp.bfloat16),
            pltpu.VMEM((2, D_MODEL, D_MODEL), jnp.bfloat16),
            pltpu.VMEM((HEADS_PER_DEV, SQ, DH), jnp.bfloat16),
            pltpu.VMEM((HEADS_PER_DEV, SQ, DH), jnp.bfloat16),
            pltpu.VMEM((HEADS_PER_DEV, SQ, DH), jnp.float32),
            pltpu.VMEM((SQ, D_MODEL), jnp.bfloat16),
            pltpu.VMEM((QBLK, D_MODEL), jnp.bfloat16),
            pltpu.VMEM((3, QBLK, KWIN), jnp.bfloat16),
            pltpu.SemaphoreType.DMA((3,)),
            pltpu.SemaphoreType.DMA((2,)),
            pltpu.SemaphoreType.DMA((3,)),
            pltpu.SemaphoreType.DMA((2,)),
            pltpu.SemaphoreType.DMA((2, HEADS_PER_DEV)),
        ],
        compiler_params=pltpu.CompilerParams(
            collective_id=0,
            vmem_limit_bytes=44 * 1024 * 1024,
        ),
    )(x_bf, wq_bf, wo_bf, K_ext, V_ext)

    return out.reshape(1, SQ, D_MODEL)
